# baseline (speedup 1.0000x reference)
import functools

import jax
import jax.numpy as jnp
from jax import lax
from jax.experimental import pallas as pl
from jax.experimental.pallas import tpu as pltpu

_LANE = 128
_TM = 1024


def _fused_lora_kernel(x_ref, w_ref, at_ref, b_ref, bias_ref, o_ref):
    xb = x_ref[...].astype(jnp.bfloat16)
    nt = (((1,), (1,)), ((), ()))

    xa = lax.dot_general(xb, at_ref[...], (((1,), (0,)), ((), ())),
                         preferred_element_type=jnp.float32)

    acc = lax.dot_general(xb, w_ref[...], nt,
                          preferred_element_type=jnp.float32)

    lora = lax.dot_general(xa.astype(jnp.bfloat16), b_ref[...], nt,
                           preferred_element_type=jnp.float32)
    o_ref[...] = acc + lora + bias_ref[...]


def kernel(x, w, bias, A, B):
    lead = x.shape[:-1]
    K = x.shape[-1]
    N = w.shape[0]
    r = A.shape[0]
    rp = -(-r // _LANE) * _LANE

    x2 = x.reshape(-1, K)
    M = x2.shape[0]
    tm = min(_TM, M)

    wb = w.astype(jnp.bfloat16)
    at = jnp.pad(A.T.astype(jnp.bfloat16), ((0, 0), (0, rp - r)))
    bb = jnp.pad(B.astype(jnp.bfloat16), ((0, 0), (0, rp - r)))
    bias2 = bias.astype(jnp.float32).reshape(1, N)

    y = pl.pallas_call(
        _fused_lora_kernel,
        out_shape=jax.ShapeDtypeStruct((M, N), jnp.float32),
        grid=(M // tm,),
        in_specs=[
            pl.BlockSpec((tm, K), lambda i: (i, 0)),
            pl.BlockSpec((N, K), lambda i: (0, 0)),
            pl.BlockSpec((K, rp), lambda i: (0, 0)),
            pl.BlockSpec((N, rp), lambda i: (0, 0)),
            pl.BlockSpec((1, N), lambda i: (0, 0)),
        ],
        out_specs=pl.BlockSpec((tm, N), lambda i: (i, 0)),
        compiler_params=pltpu.CompilerParams(
            dimension_semantics=("parallel",),
            vmem_limit_bytes=56 * 1024 * 1024,
        ),
        cost_estimate=pl.CostEstimate(
            flops=2 * M * K * N + 2 * M * K * rp + 2 * M * rp * N,
            transcendentals=0,
            bytes_accessed=(M * K + M * N) * 4 + (N * K + K * rp + N * rp) * 2,
        ),
    )(x2, wb, at, bb, bias2)
    return y.reshape(*lead, N)


# speedup vs baseline: 1.1025x; 1.1025x over previous
import functools

import jax
import jax.numpy as jnp
from jax import lax
from jax.experimental import pallas as pl
from jax.experimental.pallas import tpu as pltpu

_LANE = 128
_TM = 256


def _fused_lora_kernel(x_ref, w_ref, at_ref, b_ref, bias_ref, o_ref):
    xb = x_ref[...].astype(jnp.bfloat16)
    nt = (((1,), (1,)), ((), ()))

    xa = lax.dot_general(xb, at_ref[...], (((1,), (0,)), ((), ())),
                         preferred_element_type=jnp.float32)

    acc = lax.dot_general(xb, w_ref[...], nt,
                          preferred_element_type=jnp.float32)

    lora = lax.dot_general(xa.astype(jnp.bfloat16), b_ref[...], nt,
                           preferred_element_type=jnp.float32)
    o_ref[...] = acc + lora + bias_ref[...]


def kernel(x, w, bias, A, B):
    lead = x.shape[:-1]
    K = x.shape[-1]
    N = w.shape[0]
    r = A.shape[0]
    rp = -(-r // _LANE) * _LANE

    x2 = x.reshape(-1, K)
    M = x2.shape[0]
    tm = min(_TM, M)

    wb = w.astype(jnp.bfloat16)
    at = jnp.pad(A.T.astype(jnp.bfloat16), ((0, 0), (0, rp - r)))
    bb = jnp.pad(B.astype(jnp.bfloat16), ((0, 0), (0, rp - r)))
    bias2 = bias.astype(jnp.float32).reshape(1, N)

    y = pl.pallas_call(
        _fused_lora_kernel,
        out_shape=jax.ShapeDtypeStruct((M, N), jnp.float32),
        grid=(M // tm,),
        in_specs=[
            pl.BlockSpec((tm, K), lambda i: (i, 0)),
            pl.BlockSpec((N, K), lambda i: (0, 0)),
            pl.BlockSpec((K, rp), lambda i: (0, 0)),
            pl.BlockSpec((N, rp), lambda i: (0, 0)),
            pl.BlockSpec((1, N), lambda i: (0, 0)),
        ],
        out_specs=pl.BlockSpec((tm, N), lambda i: (i, 0)),
        compiler_params=pltpu.CompilerParams(
            dimension_semantics=("parallel",),
            vmem_limit_bytes=56 * 1024 * 1024,
        ),
        cost_estimate=pl.CostEstimate(
            flops=2 * M * K * N + 2 * M * K * rp + 2 * M * rp * N,
            transcendentals=0,
            bytes_accessed=(M * K + M * N) * 4 + (N * K + K * rp + N * rp) * 2,
        ),
    )(x2, wb, at, bb, bias2)
    return y.reshape(*lead, N)
